# baseline (speedup 1.0000x reference)
import functools

import jax
import jax.numpy as jnp
from jax import lax
from jax.experimental import pallas as pl
from jax.experimental.pallas import tpu as pltpu
from jax.experimental.pallas import tpu_sc as plsc

N = 10000
F = 128
H = F // 2
E = 320000
NC = 2
NS = 16
NW = NC * NS
NP = 10240
RPT = NP // NS
C = 125
NCHUNK = E // C
CPT = NCHUNK // NW
HCPT = CPT // 2
R = 1000
NBUF = 2

_MESH = dict(core_axis_name="c", subcore_axis_name="s", num_cores=NC,
             num_subcores=NS)


@functools.partial(
    pl.kernel,
    out_type=jax.ShapeDtypeStruct((NC * NP,), jnp.float32),
    mesh=plsc.VectorSubcoreMesh(**_MESH),
    scratch_types=[
        pltpu.VMEM((CPT, C), jnp.int32),
        pltpu.VMEM((RPT,), jnp.float32),
        pltpu.VMEM_SHARED((NP,), jnp.float32),
    ],
)
def _deg_kernel(row_hbm, out_hbm, row_v, ones_v, deg_sh):
    c = lax.axis_index("c")
    s = lax.axis_index("s")
    wid = c * NS + s
    pltpu.sync_copy(row_hbm.at[wid], row_v)

    def _fill(i, carry):
        ones_v[pl.ds(i * 16, 16)] = jnp.full((16,), 1.0, jnp.float32)
        return carry
    lax.fori_loop(0, RPT // 16, _fill, 0)
    pltpu.sync_copy(ones_v, deg_sh.at[pl.ds(s * RPT, RPT)])
    plsc.subcore_barrier()

    def _scat(j, carry):
        pltpu.sync_copy(ones_v.at[pl.ds(0, C)], deg_sh.at[row_v.at[j]],
                        add=True)
        return carry
    lax.fori_loop(0, CPT, _scat, 0)
    plsc.subcore_barrier()
    pltpu.sync_copy(deg_sh.at[pl.ds(s * RPT, RPT)],
                    out_hbm.at[pl.ds(c * NP + s * RPT, RPT)])


@functools.partial(
    pl.kernel,
    out_type=jax.ShapeDtypeStruct((NC, NP, F), jnp.float32),
    mesh=plsc.VectorSubcoreMesh(**_MESH),
    scratch_types=[
        pltpu.VMEM((HCPT, C), jnp.int32),
        pltpu.VMEM((HCPT, C), jnp.int32),
        pltpu.VMEM((NBUF, C, F), jnp.float32),
        pltpu.VMEM_SHARED((NP, F), jnp.float32),
        [pltpu.SemaphoreType.DMA] * NBUF,
        [pltpu.SemaphoreType.DMA] * NBUF,
    ],
)
def _spmm_kernel(hp_hbm, row_hbm, col_hbm, out_hbm,
                 row_v, col_v, ring, acc_sh, gsem, ssem):
    c = lax.axis_index("c")
    s = lax.axis_index("s")
    wid = c * NS + s

    buf0 = ring.at[0]

    def _zrow(i, carry):
        for k in range(F // 16):
            ring[0, i, pl.ds(k * 16, 16)] = jnp.zeros((16,), jnp.float32)
        return carry
    lax.fori_loop(0, 80, _zrow, 0)
    base = s * RPT
    for t in range(RPT // 80):
        pltpu.sync_copy(buf0.at[pl.ds(0, 80)],
                        acc_sh.at[pl.ds(base + t * 80, 80)])
    plsc.subcore_barrier()

    for half in range(2):
        pltpu.sync_copy(row_hbm.at[wid].at[pl.ds(half * HCPT, HCPT)], row_v)
        pltpu.sync_copy(col_hbm.at[wid].at[pl.ds(half * HCPT, HCPT)], col_v)
        for b in range(NBUF):
            pltpu.async_copy(hp_hbm.at[row_v.at[b]], ring.at[b], gsem[b])

        def _step(t, carry):
            j0 = NBUF * t
            for b in range(NBUF):
                pltpu.make_async_copy(hp_hbm.at[row_v.at[j0 + b]], ring.at[b],
                                      gsem[b]).wait()
                pltpu.async_copy(ring.at[b], acc_sh.at[col_v.at[j0 + b]],
                                 ssem[b], add=True)
            for b in range(NBUF):
                pltpu.make_async_copy(ring.at[b], acc_sh.at[col_v.at[j0 + b]],
                                      ssem[b]).wait()

                @pl.when(j0 + b + NBUF < HCPT)
                def _():
                    pltpu.async_copy(hp_hbm.at[row_v.at[j0 + b + NBUF]],
                                     ring.at[b], gsem[b])
            return carry
        lax.fori_loop(0, HCPT // NBUF, _step, 0)
    plsc.subcore_barrier()
    pltpu.sync_copy(acc_sh.at[pl.ds(s * RPT, RPT)],
                    out_hbm.at[c].at[pl.ds(s * RPT, RPT)])


def _mm1_body(x_ref, w_ref, b_ref, d0_ref, d1_ref,
              h_ref, hp_ref, dinv_ref):
    deg = d0_ref[0] + d1_ref[0] - 1.0
    dinv = lax.rsqrt(deg)
    h = lax.dot_general(x_ref[...], w_ref[...],
                        (((1,), (1,)), ((), ())),
                        preferred_element_type=jnp.float32) + b_ref[...]
    h_ref[...] = h
    hp_ref[...] = h * dinv
    dinv_ref[...] = dinv


def _mm2_body(p0_ref, p1_ref, h1_ref, dinv_ref, w_ref, b_ref,
              h_ref, hp_ref):
    dinv = dinv_ref[...]
    agg = p0_ref[0] + p1_ref[0]
    y = jnp.maximum(dinv * agg + (dinv * dinv) * h1_ref[...], 0.0)
    h = lax.dot_general(y, w_ref[...],
                        (((1,), (1,)), ((), ())),
                        preferred_element_type=jnp.float32) + b_ref[...]
    h_ref[...] = h
    hp_ref[...] = h * dinv


def _fin_body(q0_ref, q1_ref, h2_ref, dinv_ref, out_ref):
    dinv = dinv_ref[...]
    agg = q0_ref[0] + q1_ref[0]
    out_ref[...] = jnp.maximum(dinv * agg + (dinv * dinv) * h2_ref[...], 0.0)


def _row_spec():
    return pl.BlockSpec((R, F), lambda i: (i, 0))


def _part_spec(j):
    return pl.BlockSpec((1, R, F), lambda i, j=j: (j, i, 0))


def _dpart_spec(j):
    return pl.BlockSpec((1, R, 1), lambda i, j=j: (j, i, 0))


def _w_spec():
    return pl.BlockSpec((F, F), lambda i: (0, 0))


def _b_spec():
    return pl.BlockSpec((1, F), lambda i: (0, 0))


def _dinv_spec():
    return pl.BlockSpec((R, 1), lambda i: (i, 0))


def _mm1(xf, W1, b1, dparts3):
    return pl.pallas_call(
        _mm1_body,
        grid=(N // R,),
        in_specs=[_row_spec(), _w_spec(), _b_spec(),
                  _dpart_spec(0), _dpart_spec(1)],
        out_specs=[_row_spec(), _row_spec(), _dinv_spec()],
        out_shape=[jax.ShapeDtypeStruct((N, F), jnp.float32),
                   jax.ShapeDtypeStruct((N, F), jnp.float32),
                   jax.ShapeDtypeStruct((N, 1), jnp.float32)],
    )(xf, W1, b1, dparts3, dparts3)


def _mm2(parts, h1, dinv, W2, b2):
    return pl.pallas_call(
        _mm2_body,
        grid=(N // R,),
        in_specs=[_part_spec(0), _part_spec(1), _row_spec(), _dinv_spec(),
                  _w_spec(), _b_spec()],
        out_specs=[_row_spec(), _row_spec()],
        out_shape=[jax.ShapeDtypeStruct((N, F), jnp.float32),
                   jax.ShapeDtypeStruct((N, F), jnp.float32)],
    )(parts, parts, h1, dinv, W2, b2)


def _fin(parts, h2, dinv):
    return pl.pallas_call(
        _fin_body,
        grid=(N // R,),
        in_specs=[_part_spec(0), _part_spec(1), _row_spec(), _dinv_spec()],
        out_specs=_row_spec(),
        out_shape=jax.ShapeDtypeStruct((N, F), jnp.float32),
    )(parts, parts, h2, dinv)


@jax.jit
def _run(x, edge_index, W1, b1, W2, b2):
    xf = x[0]
    ei = edge_index.astype(jnp.int32)
    row_d = ei[0].reshape(NW, CPT, C)
    col_d = ei[1].reshape(NW, CPT, C)

    dparts = _deg_kernel(row_d)
    dparts3 = dparts.reshape(NC, NP, 1)

    h1, hp1, dinv = _mm1(xf, W1, b1.reshape(1, F), dparts3)
    parts1 = _spmm_kernel(hp1, row_d, col_d)
    h2, hp2 = _mm2(parts1, h1, dinv, W2, b2.reshape(1, F))
    parts2 = _spmm_kernel(hp2, row_d, col_d)
    out = _fin(parts2, h2, dinv)
    return out[None]


def kernel(x, edge_index, W1, b1, W2, b2):
    return _run(x, edge_index, W1, b1, W2, b2)


# speedup vs baseline: 1.0526x; 1.0526x over previous
import functools

import jax
import jax.numpy as jnp
from jax import lax
from jax.experimental import pallas as pl
from jax.experimental.pallas import tpu as pltpu
from jax.experimental.pallas import tpu_sc as plsc

N = 10000
F = 128
H = F // 2
E = 320000
NC = 2
NS = 16
NW = NC * NS
NP = 10240
RPT = NP // NS
C = 50
NCHUNK = E // C
CPT = NCHUNK // NW
NG = 8
RG = CPT // NG
R = 1000
NBUF = 5

_MESH = dict(core_axis_name="c", subcore_axis_name="s", num_cores=NC,
             num_subcores=NS)


@functools.partial(
    pl.kernel,
    out_type=jax.ShapeDtypeStruct((NC * NP,), jnp.float32),
    mesh=plsc.VectorSubcoreMesh(**_MESH),
    scratch_types=[
        pltpu.VMEM((RG, C), jnp.int32),
        pltpu.VMEM((RPT,), jnp.float32),
        pltpu.VMEM_SHARED((NP,), jnp.float32),
    ],
)
def _deg_kernel(row_hbm, out_hbm, row_v, ones_v, deg_sh):
    c = lax.axis_index("c")
    s = lax.axis_index("s")
    wid = c * NS + s

    def _fill(i, carry):
        ones_v[pl.ds(i * 16, 16)] = jnp.full((16,), 1.0, jnp.float32)
        return carry
    lax.fori_loop(0, RPT // 16, _fill, 0)
    pltpu.sync_copy(ones_v, deg_sh.at[pl.ds(s * RPT, RPT)])
    plsc.subcore_barrier()

    def _grp(g, carry):
        pltpu.sync_copy(row_hbm.at[wid, g], row_v)

        def _scat(j, carry2):
            pltpu.sync_copy(ones_v.at[pl.ds(0, C)], deg_sh.at[row_v.at[j]],
                            add=True)
            return carry2
        return lax.fori_loop(0, RG, _scat, carry)
    lax.fori_loop(0, NG, _grp, 0)
    plsc.subcore_barrier()
    pltpu.sync_copy(deg_sh.at[pl.ds(s * RPT, RPT)],
                    out_hbm.at[pl.ds(c * NP + s * RPT, RPT)])


@functools.partial(
    pl.kernel,
    out_type=jax.ShapeDtypeStruct((NC, NP, F), jnp.float32),
    mesh=plsc.VectorSubcoreMesh(**_MESH),
    scratch_types=[
        pltpu.VMEM((RG, C), jnp.int32),
        pltpu.VMEM((RG, C), jnp.int32),
        pltpu.VMEM((NBUF, C, F), jnp.float32),
        pltpu.VMEM_SHARED((NP, F), jnp.float32),
        [pltpu.SemaphoreType.DMA] * NBUF,
        [pltpu.SemaphoreType.DMA] * NBUF,
    ],
)
def _spmm_kernel(hp_hbm, row_hbm, col_hbm, out_hbm,
                 row_v, col_v, ring, acc_sh, gsem, ssem):
    c = lax.axis_index("c")
    s = lax.axis_index("s")
    wid = c * NS + s

    buf0 = ring.at[0]

    def _zrow(i, carry):
        for k in range(F // 16):
            ring[0, i, pl.ds(k * 16, 16)] = jnp.zeros((16,), jnp.float32)
        return carry
    lax.fori_loop(0, 40, _zrow, 0)
    base = s * RPT

    def _zcp(t, carry):
        pltpu.sync_copy(buf0.at[pl.ds(0, 40)],
                        acc_sh.at[pl.ds(base + t * 40, 40)])
        return carry
    lax.fori_loop(0, RPT // 40, _zcp, 0)
    plsc.subcore_barrier()

    def _grp(g, carry):
        pltpu.sync_copy(row_hbm.at[wid, g], row_v)
        pltpu.sync_copy(col_hbm.at[wid, g], col_v)
        for b in range(NBUF):
            pltpu.async_copy(hp_hbm.at[row_v.at[b]], ring.at[b], gsem[b])

        def _step(t, carry2):
            j0 = NBUF * t
            for b in range(NBUF):
                pltpu.make_async_copy(hp_hbm.at[row_v.at[j0 + b]], ring.at[b],
                                      gsem[b]).wait()
                pltpu.async_copy(ring.at[b], acc_sh.at[col_v.at[j0 + b]],
                                 ssem[b], add=True)
            for b in range(NBUF):
                pltpu.make_async_copy(ring.at[b], acc_sh.at[col_v.at[j0 + b]],
                                      ssem[b]).wait()

                @pl.when(j0 + b + NBUF < RG)
                def _():
                    pltpu.async_copy(hp_hbm.at[row_v.at[j0 + b + NBUF]],
                                     ring.at[b], gsem[b])
            return carry2
        return lax.fori_loop(0, RG // NBUF, _step, carry)
    lax.fori_loop(0, NG, _grp, 0)
    plsc.subcore_barrier()
    pltpu.sync_copy(acc_sh.at[pl.ds(s * RPT, RPT)],
                    out_hbm.at[c].at[pl.ds(s * RPT, RPT)])


def _mm1_body(x_ref, w_ref, b_ref, d0_ref, d1_ref,
              h_ref, hp_ref, dinv_ref):
    deg = d0_ref[0] + d1_ref[0] - 1.0
    dinv = lax.rsqrt(deg)
    h = lax.dot_general(x_ref[...], w_ref[...],
                        (((1,), (1,)), ((), ())),
                        preferred_element_type=jnp.float32) + b_ref[...]
    h_ref[...] = h
    hp_ref[...] = h * dinv
    dinv_ref[...] = dinv


def _mm2_body(p0_ref, p1_ref, h1_ref, dinv_ref, w_ref, b_ref,
              h_ref, hp_ref):
    dinv = dinv_ref[...]
    agg = p0_ref[0] + p1_ref[0]
    y = jnp.maximum(dinv * agg + (dinv * dinv) * h1_ref[...], 0.0)
    h = lax.dot_general(y, w_ref[...],
                        (((1,), (1,)), ((), ())),
                        preferred_element_type=jnp.float32) + b_ref[...]
    h_ref[...] = h
    hp_ref[...] = h * dinv


def _fin_body(q0_ref, q1_ref, h2_ref, dinv_ref, out_ref):
    dinv = dinv_ref[...]
    agg = q0_ref[0] + q1_ref[0]
    out_ref[...] = jnp.maximum(dinv * agg + (dinv * dinv) * h2_ref[...], 0.0)


def _row_spec():
    return pl.BlockSpec((R, F), lambda i: (i, 0))


def _part_spec(j):
    return pl.BlockSpec((1, R, F), lambda i, j=j: (j, i, 0))


def _dpart_spec(j):
    return pl.BlockSpec((1, R, 1), lambda i, j=j: (j, i, 0))


def _w_spec():
    return pl.BlockSpec((F, F), lambda i: (0, 0))


def _b_spec():
    return pl.BlockSpec((1, F), lambda i: (0, 0))


def _dinv_spec():
    return pl.BlockSpec((R, 1), lambda i: (i, 0))


def _mm1(xf, W1, b1, dparts3):
    return pl.pallas_call(
        _mm1_body,
        grid=(N // R,),
        in_specs=[_row_spec(), _w_spec(), _b_spec(),
                  _dpart_spec(0), _dpart_spec(1)],
        out_specs=[_row_spec(), _row_spec(), _dinv_spec()],
        out_shape=[jax.ShapeDtypeStruct((N, F), jnp.float32),
                   jax.ShapeDtypeStruct((N, F), jnp.float32),
                   jax.ShapeDtypeStruct((N, 1), jnp.float32)],
    )(xf, W1, b1, dparts3, dparts3)


def _mm2(parts, h1, dinv, W2, b2):
    return pl.pallas_call(
        _mm2_body,
        grid=(N // R,),
        in_specs=[_part_spec(0), _part_spec(1), _row_spec(), _dinv_spec(),
                  _w_spec(), _b_spec()],
        out_specs=[_row_spec(), _row_spec()],
        out_shape=[jax.ShapeDtypeStruct((N, F), jnp.float32),
                   jax.ShapeDtypeStruct((N, F), jnp.float32)],
    )(parts, parts, h1, dinv, W2, b2)


def _fin(parts, h2, dinv):
    return pl.pallas_call(
        _fin_body,
        grid=(N // R,),
        in_specs=[_part_spec(0), _part_spec(1), _row_spec(), _dinv_spec()],
        out_specs=_row_spec(),
        out_shape=jax.ShapeDtypeStruct((N, F), jnp.float32),
    )(parts, parts, h2, dinv)


@jax.jit
def _run(x, edge_index, W1, b1, W2, b2):
    xf = x[0]
    ei = edge_index.astype(jnp.int32)
    row_d = ei[0].reshape(NW, NG, RG, C)
    col_d = ei[1].reshape(NW, NG, RG, C)

    dparts = _deg_kernel(row_d)
    dparts3 = dparts.reshape(NC, NP, 1)

    h1, hp1, dinv = _mm1(xf, W1, b1.reshape(1, F), dparts3)
    parts1 = _spmm_kernel(hp1, row_d, col_d)
    h2, hp2 = _mm2(parts1, h1, dinv, W2, b2.reshape(1, F))
    parts2 = _spmm_kernel(hp2, row_d, col_d)
    out = _fin(parts2, h2, dinv)
    return out[None]


def kernel(x, edge_index, W1, b1, W2, b2):
    return _run(x, edge_index, W1, b1, W2, b2)


# speedup vs baseline: 1.0764x; 1.0226x over previous
import functools

import jax
import jax.numpy as jnp
from jax import lax
from jax.experimental import pallas as pl
from jax.experimental.pallas import tpu as pltpu
from jax.experimental.pallas import tpu_sc as plsc

N = 10000
F = 128
H = F // 2
E = 320000
NC = 2
NS = 16
NW = NC * NS
NP = 10240
RPT = NP // NS
C = 100
NCHUNK = E // C
CPT = NCHUNK // NW
NG = 4
RG = CPT // NG
R = 1000
NBUF = 3

_MESH = dict(core_axis_name="c", subcore_axis_name="s", num_cores=NC,
             num_subcores=NS)


@functools.partial(
    pl.kernel,
    out_type=jax.ShapeDtypeStruct((NC * NP,), jnp.float32),
    mesh=plsc.VectorSubcoreMesh(**_MESH),
    scratch_types=[
        pltpu.VMEM((RG, C), jnp.int32),
        pltpu.VMEM((RPT,), jnp.float32),
        pltpu.VMEM_SHARED((NP,), jnp.float32),
    ],
)
def _deg_kernel(row_hbm, out_hbm, row_v, ones_v, deg_sh):
    c = lax.axis_index("c")
    s = lax.axis_index("s")
    wid = c * NS + s

    def _fill(i, carry):
        ones_v[pl.ds(i * 16, 16)] = jnp.full((16,), 1.0, jnp.float32)
        return carry
    lax.fori_loop(0, RPT // 16, _fill, 0)
    pltpu.sync_copy(ones_v, deg_sh.at[pl.ds(s * RPT, RPT)])
    plsc.subcore_barrier()

    def _grp(g, carry):
        pltpu.sync_copy(row_hbm.at[wid, g], row_v)

        def _scat(j, carry2):
            pltpu.sync_copy(ones_v.at[pl.ds(0, C)], deg_sh.at[row_v.at[j]],
                            add=True)
            return carry2
        return lax.fori_loop(0, RG, _scat, carry)
    lax.fori_loop(0, NG, _grp, 0)
    plsc.subcore_barrier()
    pltpu.sync_copy(deg_sh.at[pl.ds(s * RPT, RPT)],
                    out_hbm.at[pl.ds(c * NP + s * RPT, RPT)])


@functools.partial(
    pl.kernel,
    out_type=jax.ShapeDtypeStruct((NC, NP, F), jnp.float32),
    mesh=plsc.VectorSubcoreMesh(**_MESH),
    scratch_types=[
        pltpu.VMEM((RG, C), jnp.int32),
        pltpu.VMEM((RG, C), jnp.int32),
        pltpu.VMEM((NBUF, C, F), jnp.float32),
        pltpu.VMEM_SHARED((NP, F), jnp.float32),
        [pltpu.SemaphoreType.DMA] * NBUF,
        [pltpu.SemaphoreType.DMA] * NBUF,
    ],
)
def _spmm_kernel(hp_hbm, row_hbm, col_hbm, out_hbm,
                 row_v, col_v, ring, acc_sh, gsem, ssem):
    c = lax.axis_index("c")
    s = lax.axis_index("s")
    wid = c * NS + s

    buf0 = ring.at[0]

    def _zrow(i, carry):
        for k in range(F // 16):
            ring[0, i, pl.ds(k * 16, 16)] = jnp.zeros((16,), jnp.float32)
        return carry
    lax.fori_loop(0, 40, _zrow, 0)
    base = s * RPT

    def _zcp(t, carry):
        pltpu.sync_copy(buf0.at[pl.ds(0, 40)],
                        acc_sh.at[pl.ds(base + t * 40, 40)])
        return carry
    lax.fori_loop(0, RPT // 40, _zcp, 0)
    plsc.subcore_barrier()

    def _grp(g, carry):
        pltpu.sync_copy(row_hbm.at[wid, g], row_v)
        pltpu.sync_copy(col_hbm.at[wid, g], col_v)
        for b in range(NBUF):
            pltpu.async_copy(hp_hbm.at[row_v.at[b]], ring.at[b], gsem[b])

        def _step(t, carry2):
            j0 = NBUF * t
            for b in range(NBUF):
                pltpu.make_async_copy(hp_hbm.at[row_v.at[j0 + b]], ring.at[b],
                                      gsem[b]).wait()
                pltpu.async_copy(ring.at[b], acc_sh.at[col_v.at[j0 + b]],
                                 ssem[b], add=True)
            for b in range(NBUF):
                pltpu.make_async_copy(ring.at[b], acc_sh.at[col_v.at[j0 + b]],
                                      ssem[b]).wait()

                @pl.when(j0 + b + NBUF < RG)
                def _():
                    pltpu.async_copy(hp_hbm.at[row_v.at[j0 + b + NBUF]],
                                     ring.at[b], gsem[b])
            return carry2
        carry = lax.fori_loop(0, RG // NBUF, _step, carry)
        bt = (RG - 1) % NBUF
        jt = RG - 1
        pltpu.make_async_copy(hp_hbm.at[row_v.at[jt]], ring.at[bt],
                              gsem[bt]).wait()
        pltpu.async_copy(ring.at[bt], acc_sh.at[col_v.at[jt]],
                         ssem[bt], add=True)
        pltpu.make_async_copy(ring.at[bt], acc_sh.at[col_v.at[jt]],
                              ssem[bt]).wait()
        return carry
    lax.fori_loop(0, NG, _grp, 0)
    plsc.subcore_barrier()
    pltpu.sync_copy(acc_sh.at[pl.ds(s * RPT, RPT)],
                    out_hbm.at[c].at[pl.ds(s * RPT, RPT)])


def _mm1_body(x_ref, w_ref, b_ref, d0_ref, d1_ref,
              h_ref, hp_ref, dinv_ref):
    deg = d0_ref[0] + d1_ref[0] - 1.0
    dinv = lax.rsqrt(deg)
    h = lax.dot_general(x_ref[...], w_ref[...],
                        (((1,), (1,)), ((), ())),
                        preferred_element_type=jnp.float32) + b_ref[...]
    h_ref[...] = h
    hp_ref[...] = h * dinv
    dinv_ref[...] = dinv


def _mm2_body(p0_ref, p1_ref, h1_ref, dinv_ref, w_ref, b_ref,
              h_ref, hp_ref):
    dinv = dinv_ref[...]
    agg = p0_ref[0] + p1_ref[0]
    y = jnp.maximum(dinv * agg + (dinv * dinv) * h1_ref[...], 0.0)
    h = lax.dot_general(y, w_ref[...],
                        (((1,), (1,)), ((), ())),
                        preferred_element_type=jnp.float32) + b_ref[...]
    h_ref[...] = h
    hp_ref[...] = h * dinv


def _fin_body(q0_ref, q1_ref, h2_ref, dinv_ref, out_ref):
    dinv = dinv_ref[...]
    agg = q0_ref[0] + q1_ref[0]
    out_ref[...] = jnp.maximum(dinv * agg + (dinv * dinv) * h2_ref[...], 0.0)


def _row_spec():
    return pl.BlockSpec((R, F), lambda i: (i, 0))


def _part_spec(j):
    return pl.BlockSpec((1, R, F), lambda i, j=j: (j, i, 0))


def _dpart_spec(j):
    return pl.BlockSpec((1, R, 1), lambda i, j=j: (j, i, 0))


def _w_spec():
    return pl.BlockSpec((F, F), lambda i: (0, 0))


def _b_spec():
    return pl.BlockSpec((1, F), lambda i: (0, 0))


def _dinv_spec():
    return pl.BlockSpec((R, 1), lambda i: (i, 0))


def _mm1(xf, W1, b1, dparts3):
    return pl.pallas_call(
        _mm1_body,
        grid=(N // R,),
        in_specs=[_row_spec(), _w_spec(), _b_spec(),
                  _dpart_spec(0), _dpart_spec(1)],
        out_specs=[_row_spec(), _row_spec(), _dinv_spec()],
        out_shape=[jax.ShapeDtypeStruct((N, F), jnp.float32),
                   jax.ShapeDtypeStruct((N, F), jnp.float32),
                   jax.ShapeDtypeStruct((N, 1), jnp.float32)],
    )(xf, W1, b1, dparts3, dparts3)


def _mm2(parts, h1, dinv, W2, b2):
    return pl.pallas_call(
        _mm2_body,
        grid=(N // R,),
        in_specs=[_part_spec(0), _part_spec(1), _row_spec(), _dinv_spec(),
                  _w_spec(), _b_spec()],
        out_specs=[_row_spec(), _row_spec()],
        out_shape=[jax.ShapeDtypeStruct((N, F), jnp.float32),
                   jax.ShapeDtypeStruct((N, F), jnp.float32)],
    )(parts, parts, h1, dinv, W2, b2)


def _fin(parts, h2, dinv):
    return pl.pallas_call(
        _fin_body,
        grid=(N // R,),
        in_specs=[_part_spec(0), _part_spec(1), _row_spec(), _dinv_spec()],
        out_specs=_row_spec(),
        out_shape=jax.ShapeDtypeStruct((N, F), jnp.float32),
    )(parts, parts, h2, dinv)


@jax.jit
def _run(x, edge_index, W1, b1, W2, b2):
    xf = x[0]
    ei = edge_index.astype(jnp.int32)
    row_d = ei[0].reshape(NW, NG, RG, C)
    col_d = ei[1].reshape(NW, NG, RG, C)

    dparts = _deg_kernel(row_d)
    dparts3 = dparts.reshape(NC, NP, 1)

    h1, hp1, dinv = _mm1(xf, W1, b1.reshape(1, F), dparts3)
    parts1 = _spmm_kernel(hp1, row_d, col_d)
    h2, hp2 = _mm2(parts1, h1, dinv, W2, b2.reshape(1, F))
    parts2 = _spmm_kernel(hp2, row_d, col_d)
    out = _fin(parts2, h2, dinv)
    return out[None]


def kernel(x, edge_index, W1, b1, W2, b2):
    return _run(x, edge_index, W1, b1, W2, b2)


# speedup vs baseline: 1.1100x; 1.0313x over previous
import functools

import jax
import jax.numpy as jnp
from jax import lax
from jax.experimental import pallas as pl
from jax.experimental.pallas import tpu as pltpu
from jax.experimental.pallas import tpu_sc as plsc

N = 10000
F = 128
H = F // 2
E = 320000
NC = 2
NS = 16
NW = NC * NS
NP = 10112
RPT = NP // NS
NPD = 10240
RPTD = NPD // NS
C = 80
NCHUNK = E // C
CPT = NCHUNK // NW
NG = 5
RG = CPT // NG
R = 1000
NBUF = 4

_MESH = dict(core_axis_name="c", subcore_axis_name="s", num_cores=NC,
             num_subcores=NS)


@functools.partial(
    pl.kernel,
    out_type=jax.ShapeDtypeStruct((NC * NPD,), jnp.float32),
    mesh=plsc.VectorSubcoreMesh(**_MESH),
    scratch_types=[
        pltpu.VMEM((RG, C), jnp.int32),
        pltpu.VMEM((RPTD,), jnp.float32),
        pltpu.VMEM_SHARED((NPD,), jnp.float32),
    ],
)
def _deg_kernel(row_hbm, out_hbm, row_v, ones_v, deg_sh):
    c = lax.axis_index("c")
    s = lax.axis_index("s")
    wid = c * NS + s

    def _fill(i, carry):
        ones_v[pl.ds(i * 16, 16)] = jnp.full((16,), 1.0, jnp.float32)
        return carry
    lax.fori_loop(0, RPTD // 16, _fill, 0)
    pltpu.sync_copy(ones_v, deg_sh.at[pl.ds(s * RPTD, RPTD)])
    plsc.subcore_barrier()

    def _grp(g, carry):
        pltpu.sync_copy(row_hbm.at[wid, g], row_v)

        def _scat(j, carry2):
            pltpu.sync_copy(ones_v.at[pl.ds(0, C)], deg_sh.at[row_v.at[j]],
                            add=True)
            return carry2
        return lax.fori_loop(0, RG, _scat, carry)
    lax.fori_loop(0, NG, _grp, 0)
    plsc.subcore_barrier()
    pltpu.sync_copy(deg_sh.at[pl.ds(s * RPTD, RPTD)],
                    out_hbm.at[pl.ds(c * NPD + s * RPTD, RPTD)])


@functools.partial(
    pl.kernel,
    out_type=jax.ShapeDtypeStruct((NC, NP, F), jnp.float32),
    mesh=plsc.VectorSubcoreMesh(**_MESH),
    scratch_types=[
        pltpu.VMEM((RG, C), jnp.int32),
        pltpu.VMEM((RG, C), jnp.int32),
        pltpu.VMEM((NBUF, C, F), jnp.float32),
        pltpu.VMEM_SHARED((NP, F), jnp.float32),
        [pltpu.SemaphoreType.DMA] * NBUF,
        [pltpu.SemaphoreType.DMA] * NBUF,
    ],
)
def _spmm_kernel(hp_hbm, row_hbm, col_hbm, out_hbm,
                 row_v, col_v, ring, acc_sh, gsem, ssem):
    c = lax.axis_index("c")
    s = lax.axis_index("s")
    wid = c * NS + s

    buf0 = ring.at[0]

    def _zrow(i, carry):
        for k in range(F // 16):
            ring[0, i, pl.ds(k * 16, 16)] = jnp.zeros((16,), jnp.float32)
        return carry
    lax.fori_loop(0, 40, _zrow, 0)
    base = s * RPT

    def _zcp(t, carry):
        pltpu.sync_copy(buf0.at[pl.ds(0, 40)],
                        acc_sh.at[pl.ds(base + t * 40, 40)])
        return carry
    lax.fori_loop(0, RPT // 40, _zcp, 0)
    zrem = RPT - (RPT // 40) * 40
    if zrem:
        pltpu.sync_copy(buf0.at[pl.ds(0, zrem)],
                        acc_sh.at[pl.ds(base + RPT - zrem, zrem)])
    plsc.subcore_barrier()

    def _grp(g, carry):
        pltpu.sync_copy(row_hbm.at[wid, g], row_v)
        pltpu.sync_copy(col_hbm.at[wid, g], col_v)
        for b in range(NBUF):
            pltpu.async_copy(hp_hbm.at[row_v.at[b]], ring.at[b], gsem[b])

        def _step(t, carry2):
            j0 = NBUF * t
            for b in range(NBUF):
                pltpu.make_async_copy(hp_hbm.at[row_v.at[j0 + b]], ring.at[b],
                                      gsem[b]).wait()
                pltpu.async_copy(ring.at[b], acc_sh.at[col_v.at[j0 + b]],
                                 ssem[b], add=True)
            for b in range(NBUF):
                pltpu.make_async_copy(ring.at[b], acc_sh.at[col_v.at[j0 + b]],
                                      ssem[b]).wait()

                @pl.when(j0 + b + NBUF < RG)
                def _():
                    pltpu.async_copy(hp_hbm.at[row_v.at[j0 + b + NBUF]],
                                     ring.at[b], gsem[b])
            return carry2
        carry = lax.fori_loop(0, RG // NBUF, _step, carry)
        bt = (RG - 1) % NBUF
        jt = RG - 1
        pltpu.make_async_copy(hp_hbm.at[row_v.at[jt]], ring.at[bt],
                              gsem[bt]).wait()
        pltpu.async_copy(ring.at[bt], acc_sh.at[col_v.at[jt]],
                         ssem[bt], add=True)
        pltpu.make_async_copy(ring.at[bt], acc_sh.at[col_v.at[jt]],
                              ssem[bt]).wait()
        return carry
    lax.fori_loop(0, NG, _grp, 0)
    plsc.subcore_barrier()
    pltpu.sync_copy(acc_sh.at[pl.ds(s * RPT, RPT)],
                    out_hbm.at[c].at[pl.ds(s * RPT, RPT)])


def _mm1_body(x_ref, w_ref, b_ref, d0_ref, d1_ref,
              h_ref, hp_ref, dinv_ref):
    deg = d0_ref[0] + d1_ref[0] - 1.0
    dinv = lax.rsqrt(deg)
    h = lax.dot_general(x_ref[...], w_ref[...],
                        (((1,), (1,)), ((), ())),
                        preferred_element_type=jnp.float32) + b_ref[...]
    h_ref[...] = h
    hp_ref[...] = h * dinv
    dinv_ref[...] = dinv


def _mm2_body(p0_ref, p1_ref, h1_ref, dinv_ref, w_ref, b_ref,
              h_ref, hp_ref):
    dinv = dinv_ref[...]
    agg = p0_ref[0] + p1_ref[0]
    y = jnp.maximum(dinv * agg + (dinv * dinv) * h1_ref[...], 0.0)
    h = lax.dot_general(y, w_ref[...],
                        (((1,), (1,)), ((), ())),
                        preferred_element_type=jnp.float32) + b_ref[...]
    h_ref[...] = h
    hp_ref[...] = h * dinv


def _fin_body(q0_ref, q1_ref, h2_ref, dinv_ref, out_ref):
    dinv = dinv_ref[...]
    agg = q0_ref[0] + q1_ref[0]
    out_ref[...] = jnp.maximum(dinv * agg + (dinv * dinv) * h2_ref[...], 0.0)


def _row_spec():
    return pl.BlockSpec((R, F), lambda i: (i, 0))


def _part_spec(j):
    return pl.BlockSpec((1, R, F), lambda i, j=j: (j, i, 0))


def _dpart_spec(j):
    return pl.BlockSpec((1, R, 1), lambda i, j=j: (j, i, 0))


def _w_spec():
    return pl.BlockSpec((F, F), lambda i: (0, 0))


def _b_spec():
    return pl.BlockSpec((1, F), lambda i: (0, 0))


def _dinv_spec():
    return pl.BlockSpec((R, 1), lambda i: (i, 0))


def _mm1(xf, W1, b1, dparts3):
    return pl.pallas_call(
        _mm1_body,
        grid=(N // R,),
        in_specs=[_row_spec(), _w_spec(), _b_spec(),
                  _dpart_spec(0), _dpart_spec(1)],
        out_specs=[_row_spec(), _row_spec(), _dinv_spec()],
        out_shape=[jax.ShapeDtypeStruct((N, F), jnp.float32),
                   jax.ShapeDtypeStruct((N, F), jnp.float32),
                   jax.ShapeDtypeStruct((N, 1), jnp.float32)],
    )(xf, W1, b1, dparts3, dparts3)


def _mm2(parts, h1, dinv, W2, b2):
    return pl.pallas_call(
        _mm2_body,
        grid=(N // R,),
        in_specs=[_part_spec(0), _part_spec(1), _row_spec(), _dinv_spec(),
                  _w_spec(), _b_spec()],
        out_specs=[_row_spec(), _row_spec()],
        out_shape=[jax.ShapeDtypeStruct((N, F), jnp.float32),
                   jax.ShapeDtypeStruct((N, F), jnp.float32)],
    )(parts, parts, h1, dinv, W2, b2)


def _fin(parts, h2, dinv):
    return pl.pallas_call(
        _fin_body,
        grid=(N // R,),
        in_specs=[_part_spec(0), _part_spec(1), _row_spec(), _dinv_spec()],
        out_specs=_row_spec(),
        out_shape=jax.ShapeDtypeStruct((N, F), jnp.float32),
    )(parts, parts, h2, dinv)


@jax.jit
def _run(x, edge_index, W1, b1, W2, b2):
    xf = x[0]
    ei = edge_index.astype(jnp.int32)
    row_d = ei[0].reshape(NW, NG, RG, C)
    col_d = ei[1].reshape(NW, NG, RG, C)

    dparts = _deg_kernel(row_d)
    dparts3 = dparts.reshape(NC, NPD, 1)

    h1, hp1, dinv = _mm1(xf, W1, b1.reshape(1, F), dparts3)
    parts1 = _spmm_kernel(hp1, row_d, col_d)
    h2, hp2 = _mm2(parts1, h1, dinv, W2, b2.reshape(1, F))
    parts2 = _spmm_kernel(hp2, row_d, col_d)
    out = _fin(parts2, h2, dinv)
    return out[None]


def kernel(x, edge_index, W1, b1, W2, b2):
    return _run(x, edge_index, W1, b1, W2, b2)


# speedup vs baseline: 1.1494x; 1.0355x over previous
import functools

import jax
import jax.numpy as jnp
from jax import lax
from jax.experimental import pallas as pl
from jax.experimental.pallas import tpu as pltpu
from jax.experimental.pallas import tpu_sc as plsc

N = 10000
F = 128
H = F // 2
E = 320000
NC = 2
NS = 16
NW = NC * NS
NP = 10112
RPT = NP // NS
NPD = 10240
RPTD = NPD // NS
C = 80
NCHUNK = E // C
CPT = NCHUNK // NW
NG = 5
RG = CPT // NG
R = 1000
NBUF = 4

_MESH = dict(core_axis_name="c", subcore_axis_name="s", num_cores=NC,
             num_subcores=NS)


@functools.partial(
    pl.kernel,
    out_type=jax.ShapeDtypeStruct((NPD,), jnp.float32),
    mesh=plsc.VectorSubcoreMesh(**_MESH),
    scratch_types=[
        pltpu.VMEM((RG, C), jnp.int32),
        pltpu.VMEM((RPTD,), jnp.float32),
        pltpu.VMEM_SHARED((NPD,), jnp.float32),
        pltpu.SemaphoreType.DMA,
    ],
)
def _deg_kernel(row_hbm, out_hbm, row_v, ones_v, deg_sh, sem):
    c = lax.axis_index("c")
    s = lax.axis_index("s")

    def _fill(i, carry):
        ones_v[pl.ds(i * 16, 16)] = jnp.full((16,), 1.0, jnp.float32)
        return carry
    lax.fori_loop(0, RPTD // 16, _fill, 0)
    pltpu.sync_copy(ones_v, deg_sh.at[pl.ds(s * RPTD, RPTD)])
    plsc.subcore_barrier()

    ones_c = ones_v.at[pl.ds(0, C)]
    for k in range(2):
        def _grp(g, carry):
            pltpu.sync_copy(row_hbm.at[2 * s + k, g], row_v)

            def _fire(j, carry2):
                pltpu.async_copy(ones_c, deg_sh.at[row_v.at[j]], sem,
                                 add=True)
                return carry2
            lax.fori_loop(0, RG, _fire, carry)

            def _drain(j, carry2):
                pltpu.make_async_copy(ones_c, deg_sh.at[row_v.at[j]],
                                      sem).wait()
                return carry2
            return lax.fori_loop(0, RG, _drain, carry)
        lax.fori_loop(0, NG, _grp, 0)
    plsc.subcore_barrier()

    @pl.when(c == 0)
    def _():
        pltpu.sync_copy(deg_sh.at[pl.ds(s * RPTD, RPTD)],
                        out_hbm.at[pl.ds(s * RPTD, RPTD)])


@functools.partial(
    pl.kernel,
    out_type=jax.ShapeDtypeStruct((NC, NP, F), jnp.float32),
    mesh=plsc.VectorSubcoreMesh(**_MESH),
    scratch_types=[
        pltpu.VMEM((RG, C), jnp.int32),
        pltpu.VMEM((RG, C), jnp.int32),
        pltpu.VMEM((NBUF, C, F), jnp.float32),
        pltpu.VMEM_SHARED((NP, F), jnp.float32),
        [pltpu.SemaphoreType.DMA] * NBUF,
        [pltpu.SemaphoreType.DMA] * NBUF,
    ],
)
def _spmm_kernel(hp_hbm, row_hbm, col_hbm, out_hbm,
                 row_v, col_v, ring, acc_sh, gsem, ssem):
    c = lax.axis_index("c")
    s = lax.axis_index("s")
    wid = c * NS + s

    buf0 = ring.at[0]

    def _zrow(i, carry):
        for k in range(F // 16):
            ring[0, i, pl.ds(k * 16, 16)] = jnp.zeros((16,), jnp.float32)
        return carry
    lax.fori_loop(0, 40, _zrow, 0)
    base = s * RPT

    def _zcp(t, carry):
        pltpu.sync_copy(buf0.at[pl.ds(0, 40)],
                        acc_sh.at[pl.ds(base + t * 40, 40)])
        return carry
    lax.fori_loop(0, RPT // 40, _zcp, 0)
    zrem = RPT - (RPT // 40) * 40
    if zrem:
        pltpu.sync_copy(buf0.at[pl.ds(0, zrem)],
                        acc_sh.at[pl.ds(base + RPT - zrem, zrem)])
    plsc.subcore_barrier()

    def _grp(g, carry):
        pltpu.sync_copy(row_hbm.at[wid, g], row_v)
        pltpu.sync_copy(col_hbm.at[wid, g], col_v)
        for b in range(NBUF):
            pltpu.async_copy(hp_hbm.at[row_v.at[b]], ring.at[b], gsem[b])

        def _step(t, carry2):
            j0 = NBUF * t
            for b in range(NBUF):
                pltpu.make_async_copy(hp_hbm.at[row_v.at[j0 + b]], ring.at[b],
                                      gsem[b]).wait()
                pltpu.async_copy(ring.at[b], acc_sh.at[col_v.at[j0 + b]],
                                 ssem[b], add=True)
            for b in range(NBUF):
                pltpu.make_async_copy(ring.at[b], acc_sh.at[col_v.at[j0 + b]],
                                      ssem[b]).wait()

                @pl.when(j0 + b + NBUF < RG)
                def _():
                    pltpu.async_copy(hp_hbm.at[row_v.at[j0 + b + NBUF]],
                                     ring.at[b], gsem[b])
            return carry2
        carry = lax.fori_loop(0, RG // NBUF, _step, carry)
        bt = (RG - 1) % NBUF
        jt = RG - 1
        pltpu.make_async_copy(hp_hbm.at[row_v.at[jt]], ring.at[bt],
                              gsem[bt]).wait()
        pltpu.async_copy(ring.at[bt], acc_sh.at[col_v.at[jt]],
                         ssem[bt], add=True)
        pltpu.make_async_copy(ring.at[bt], acc_sh.at[col_v.at[jt]],
                              ssem[bt]).wait()
        return carry
    lax.fori_loop(0, NG, _grp, 0)
    plsc.subcore_barrier()
    pltpu.sync_copy(acc_sh.at[pl.ds(s * RPT, RPT)],
                    out_hbm.at[c].at[pl.ds(s * RPT, RPT)])


def _mm1_body(x_ref, w_ref, b_ref, d_ref, h_ref, hp_ref):
    dinv = lax.rsqrt(d_ref[...])
    h = lax.dot_general(x_ref[...], w_ref[...],
                        (((1,), (1,)), ((), ())),
                        preferred_element_type=jnp.float32) + b_ref[...]
    h_ref[...] = h
    hp_ref[...] = h * dinv


def _mm2_body(p0_ref, p1_ref, h1_ref, d_ref, w_ref, b_ref,
              h_ref, hp_ref):
    dinv = lax.rsqrt(d_ref[...])
    agg = p0_ref[0] + p1_ref[0]
    y = jnp.maximum(dinv * agg + (dinv * dinv) * h1_ref[...], 0.0)
    h = lax.dot_general(y, w_ref[...],
                        (((1,), (1,)), ((), ())),
                        preferred_element_type=jnp.float32) + b_ref[...]
    h_ref[...] = h
    hp_ref[...] = h * dinv


def _fin_body(q0_ref, q1_ref, h2_ref, d_ref, out_ref):
    dinv = lax.rsqrt(d_ref[...])
    agg = q0_ref[0] + q1_ref[0]
    out_ref[...] = jnp.maximum(dinv * agg + (dinv * dinv) * h2_ref[...], 0.0)


def _row_spec():
    return pl.BlockSpec((R, F), lambda i: (i, 0))


def _part_spec(j):
    return pl.BlockSpec((1, R, F), lambda i, j=j: (j, i, 0))


def _deg_spec():
    return pl.BlockSpec((R, 1), lambda i: (i, 0))


def _w_spec():
    return pl.BlockSpec((F, F), lambda i: (0, 0))


def _b_spec():
    return pl.BlockSpec((1, F), lambda i: (0, 0))


def _mm1(xf, W1, b1, degb):
    return pl.pallas_call(
        _mm1_body,
        grid=(N // R,),
        in_specs=[_row_spec(), _w_spec(), _b_spec(), _deg_spec()],
        out_specs=[_row_spec(), _row_spec()],
        out_shape=[jax.ShapeDtypeStruct((N, F), jnp.float32),
                   jax.ShapeDtypeStruct((N, F), jnp.float32)],
    )(xf, W1, b1, degb)


def _mm2(parts, h1, degb, W2, b2):
    return pl.pallas_call(
        _mm2_body,
        grid=(N // R,),
        in_specs=[_part_spec(0), _part_spec(1), _row_spec(), _deg_spec(),
                  _w_spec(), _b_spec()],
        out_specs=[_row_spec(), _row_spec()],
        out_shape=[jax.ShapeDtypeStruct((N, F), jnp.float32),
                   jax.ShapeDtypeStruct((N, F), jnp.float32)],
    )(parts, parts, h1, degb, W2, b2)


def _fin(parts, h2, degb):
    return pl.pallas_call(
        _fin_body,
        grid=(N // R,),
        in_specs=[_part_spec(0), _part_spec(1), _row_spec(), _deg_spec()],
        out_specs=_row_spec(),
        out_shape=jax.ShapeDtypeStruct((N, F), jnp.float32),
    )(parts, parts, h2, degb)


@jax.jit
def _run(x, edge_index, W1, b1, W2, b2):
    xf = x[0]
    ei = edge_index.astype(jnp.int32)
    row_d = ei[0].reshape(NW, NG, RG, C)
    col_d = ei[1].reshape(NW, NG, RG, C)

    degb = _deg_kernel(row_d).reshape(NPD, 1)

    h1, hp1 = _mm1(xf, W1, b1.reshape(1, F), degb)
    parts1 = _spmm_kernel(hp1, row_d, col_d)
    h2, hp2 = _mm2(parts1, h1, degb, W2, b2.reshape(1, F))
    parts2 = _spmm_kernel(hp2, row_d, col_d)
    out = _fin(parts2, h2, degb)
    return out[None]


def kernel(x, edge_index, W1, b1, W2, b2):
    return _run(x, edge_index, W1, b1, W2, b2)
